# baseline (speedup 1.0000x reference)
import functools

import jax
import jax.numpy as jnp
from jax import lax
from jax.experimental import pallas as pl
from jax.experimental.pallas import tpu as pltpu
from jax.experimental.pallas import tpu_sc as plsc

E = 8
D = 2048
DFF = 4096
T = 2048

BLK = 256
G = (2 * T) // BLK + E
GB = G * BLK
F_TILE = 1024
NF = DFF // F_TILE

NW = 32
ROWS_PER_W = GB // NW
CH = 24
NCH = ROWS_PER_W // CH
T_PER_W = T // NW
CH2 = 16


def _routing_kernel(x_ref, wg_ref, slot_ref, probs_ref, be_ref):
    logits = jax.lax.dot(x_ref[...], wg_ref[...],
                         preferred_element_type=jnp.float32)
    gates = jax.nn.softmax(logits, axis=1)
    iota = jax.lax.broadcasted_iota(jnp.int32, (T, E), 1)
    m1 = jnp.max(gates, axis=1, keepdims=True)
    a1 = jnp.min(jnp.where(gates == m1, iota, E), axis=1, keepdims=True)
    g2 = jnp.where(iota == a1, -jnp.inf, gates)
    m2 = jnp.max(g2, axis=1, keepdims=True)
    a2 = jnp.min(jnp.where(g2 == m2, iota, E), axis=1, keepdims=True)
    denom = jnp.maximum(m1 + m2, 1.1920929e-07)
    probs_ref[...] = jnp.concatenate([m1 / denom, m2 / denom], axis=1)

    mask1 = (iota == a1).astype(jnp.bfloat16)
    mask2 = (iota == a2).astype(jnp.bfloat16)
    r_i = jax.lax.broadcasted_iota(jnp.int32, (T, T), 0)
    c_i = jax.lax.broadcasted_iota(jnp.int32, (T, T), 1)
    tri = (c_i <= r_i).astype(jnp.bfloat16)
    c12 = jax.lax.dot(tri, jnp.concatenate([mask1, mask2], axis=1),
                      preferred_element_type=jnp.float32)
    c1, c2 = c12[:, :E], c12[:, E:]
    m1f = mask1.astype(jnp.float32)
    m2f = mask2.astype(jnp.float32)
    total1 = jnp.sum(m1f, axis=0, keepdims=True)
    total2 = jnp.sum(m2f, axis=0, keepdims=True)
    counts = total1 + total2
    pblocks = jnp.floor((counts + (BLK - 1)) * (1.0 / BLK))
    e_r = jax.lax.broadcasted_iota(jnp.int32, (E, E), 0)
    e_c = jax.lax.broadcasted_iota(jnp.int32, (E, E), 1)
    u8 = (e_r <= e_c).astype(jnp.float32)
    pend = jax.lax.dot(jnp.broadcast_to(pblocks, (E, E)), u8,
                       precision=jax.lax.Precision.HIGHEST,
                       preferred_element_type=jnp.float32)[0:1, :]
    offs = (pend - pblocks) * float(BLK)
    slot1 = jnp.sum((offs + c1 - 1.0) * m1f, axis=1, keepdims=True)
    slot2 = jnp.sum((offs + total1 + c2 - 1.0) * m2f, axis=1, keepdims=True)
    slot_ref[...] = jnp.concatenate([slot1, slot2], axis=1).astype(jnp.int32)

    b_i = jax.lax.broadcasted_iota(jnp.int32, (G, E), 0).astype(jnp.float32)
    pend_b = jnp.broadcast_to(pend, (G, E))
    be = jnp.sum((pend_b <= b_i).astype(jnp.float32), axis=1, keepdims=True)
    be_ref[...] = jnp.minimum(be, float(E - 1)).astype(jnp.int32)


def _routing(x, wg):
    return pl.pallas_call(
        _routing_kernel,
        out_shape=(jax.ShapeDtypeStruct((T, 2), jnp.int32),
                   jax.ShapeDtypeStruct((T, 2), jnp.float32),
                   jax.ShapeDtypeStruct((G, 1), jnp.int32)),
    )(x, wg)


def _build_indices(slot, probs, be):
    slot_flat = slot.reshape(-1)
    tokens = (jnp.arange(2 * T, dtype=jnp.int32) // 2)
    row_token = jnp.zeros((GB,), jnp.int32).at[slot_flat].set(tokens)
    row_w = jnp.zeros((GB,), jnp.float32).at[slot_flat].set(probs.reshape(-1))
    return row_token, row_w, be.reshape(G), slot[:, 0], slot[:, 1]


def _dispatch(x, row_token):
    mesh = plsc.VectorSubcoreMesh(core_axis_name="c", subcore_axis_name="s")

    @functools.partial(
        pl.kernel, mesh=mesh,
        out_type=jax.ShapeDtypeStruct((GB, D), jnp.float32),
        scratch_types=[
            pltpu.VMEM((ROWS_PER_W,), jnp.int32),
            pltpu.VMEM((CH, D), jnp.float32),
            pltpu.VMEM((CH, D), jnp.float32),
            pltpu.SemaphoreType.DMA,
            pltpu.SemaphoreType.DMA,
            pltpu.SemaphoreType.DMA,
            pltpu.SemaphoreType.DMA,
        ],
    )
    def k(x_hbm, idx_hbm, out_hbm, idx_v, r0, r1, sg0, sg1, sw0, sw1):
        wid = lax.axis_index("s") * 2 + lax.axis_index("c")
        base = wid * ROWS_PER_W
        pltpu.sync_copy(idx_hbm.at[pl.ds(base, ROWS_PER_W)], idx_v)
        bufs = [(r0, sg0, sw0), (r1, sg1, sw1)]
        gh = [None, None]
        wh = [None, None]
        gh[0] = pltpu.async_copy(
            x_hbm.at[idx_v.at[pl.ds(0, CH)]], bufs[0][0], bufs[0][1])
        for c in range(NCH):
            cur = c % 2
            nxt = (c + 1) % 2
            if c + 1 < NCH:
                if wh[nxt] is not None:
                    wh[nxt].wait()
                gh[nxt] = pltpu.async_copy(
                    x_hbm.at[idx_v.at[pl.ds((c + 1) * CH, CH)]],
                    bufs[nxt][0], bufs[nxt][1])
            gh[cur].wait()
            wh[cur] = pltpu.async_copy(
                bufs[cur][0], out_hbm.at[pl.ds(base + c * CH, CH)],
                bufs[cur][2])
        wh[0].wait()
        wh[1].wait()

    return k(x, row_token)


def _ffn_kernel(be_ref, xs_ref, rw_ref, wg_ref, wu_ref, wd_ref, out_ref):
    f = pl.program_id(1)
    xb = xs_ref[...].astype(jnp.bfloat16)
    h = jax.lax.dot(xb, wg_ref[0], preferred_element_type=jnp.float32)
    u = jax.lax.dot(xb, wu_ref[0], preferred_element_type=jnp.float32)
    inter = (jax.nn.silu(h) * u * rw_ref[...]).astype(jnp.bfloat16)
    contrib = jax.lax.dot(inter, wd_ref[0], preferred_element_type=jnp.float32)

    @pl.when(f == 0)
    def _init():
        out_ref[...] = contrib

    @pl.when(f > 0)
    def _acc():
        out_ref[...] += contrib


def _ffn(block_expert, xs, row_w, wgb, wub, wdb):
    grid_spec = pltpu.PrefetchScalarGridSpec(
        num_scalar_prefetch=1,
        grid=(G, NF),
        in_specs=[
            pl.BlockSpec((BLK, D), lambda g, f, be: (g, 0)),
            pl.BlockSpec((BLK, 1), lambda g, f, be: (g, 0)),
            pl.BlockSpec((1, D, F_TILE), lambda g, f, be: (be[g], 0, f)),
            pl.BlockSpec((1, D, F_TILE), lambda g, f, be: (be[g], 0, f)),
            pl.BlockSpec((1, F_TILE, D), lambda g, f, be: (be[g], f, 0)),
        ],
        out_specs=pl.BlockSpec((BLK, D), lambda g, f, be: (g, 0)),
    )
    return pl.pallas_call(
        _ffn_kernel,
        grid_spec=grid_spec,
        out_shape=jax.ShapeDtypeStruct((GB, D), jnp.float32),
    )(block_expert, xs, row_w.reshape(GB, 1), wgb, wub, wdb)


def _combine(ys, pos0, pos1):
    mesh = plsc.VectorSubcoreMesh(core_axis_name="c", subcore_axis_name="s")

    @functools.partial(
        pl.kernel, mesh=mesh,
        out_type=jax.ShapeDtypeStruct((T, D), jnp.float32),
        scratch_types=[
            pltpu.VMEM((CH2,), jnp.int32),
            pltpu.VMEM((CH2,), jnp.int32),
            pltpu.VMEM((CH2, D), jnp.float32),
            pltpu.VMEM((CH2, D), jnp.float32),
            pltpu.SemaphoreType.DMA,
            pltpu.SemaphoreType.DMA,
        ],
    )
    def k(ys_hbm, p0_hbm, p1_hbm, out_hbm, i0_v, i1_v, r0_v, r1_v, s0, s1):
        wid = lax.axis_index("s") * 2 + lax.axis_index("c")

        @pl.loop(0, T_PER_W // CH2)
        def _(c):
            base = wid * T_PER_W + c * CH2
            pltpu.sync_copy(p0_hbm.at[pl.ds(base, CH2)], i0_v)
            pltpu.sync_copy(p1_hbm.at[pl.ds(base, CH2)], i1_v)
            cp0 = pltpu.async_copy(ys_hbm.at[i0_v], r0_v, s0)
            cp1 = pltpu.async_copy(ys_hbm.at[i1_v], r1_v, s1)
            cp0.wait()
            cp1.wait()

            @pl.loop(0, CH2)
            def _(r):
                @pl.loop(0, D // 16)
                def _(j):
                    sl = pl.ds(j * 16, 16)
                    r0_v.at[r, sl][...] = r0_v.at[r, sl][...] + r1_v.at[r, sl][...]

            pltpu.sync_copy(r0_v, out_hbm.at[pl.ds(base, CH2)])

    return k(ys, pos0, pos1)


@jax.jit
def kernel(x, wg, w_gate, w_up, w_down):
    slot, probs, be = _routing(x, wg)
    row_token, row_w, block_expert, pos0, pos1 = _build_indices(slot, probs, be)
    xs = _dispatch(x, row_token)
    wgb = w_gate.astype(jnp.bfloat16)
    wub = w_up.astype(jnp.bfloat16)
    wdb = w_down.astype(jnp.bfloat16)
    ys = _ffn(block_expert, xs, row_w, wgb, wub, wdb)
    return _combine(ys, pos0, pos1)
